# baseline (speedup 1.0000x reference)
import functools

import jax
import jax.numpy as jnp
from jax import lax
from jax.experimental import pallas as pl
from jax.experimental.pallas import tpu as pltpu
from jax.experimental.pallas import tpu_sc as plsc

N = 10000
E = 160000
D = 256
H = 2048
HID = 128
C = 40

W = 384
NC = 2
NS = 16
NW = NC * NS
HPW = H // NW
NV = 10240
NB16 = E // 16


@functools.lru_cache(maxsize=None)
def _sc_mesh():
    return plsc.VectorSubcoreMesh(core_axis_name="c", subcore_axis_name="s",
                                  num_cores=NC, num_subcores=NS)


_IOTA16 = None


def _extract(v, nbits):
    i = lax.iota(jnp.int32, 16)
    lane0 = i == 0
    acc = jnp.int32(0)
    for k in range(nbits):
        bit = jnp.any(lane0 & (((v >> k) & 1) == 1))
        acc = acc + jnp.where(bit, jnp.int32(1 << k), jnp.int32(0))
    return acc


def _build_b_body(nidx_hbm, hidx_hbm, zrow_hbm, b_out,
                  nidx_v, hidx_v, probe_v, row_v):
    c = lax.axis_index("c")
    s = lax.axis_index("s")
    w = c * NS + s
    iota16 = lax.iota(jnp.int32, 16)

    def first_geq(target):
        def step(_, carry):
            lo_b, hi_b = carry
            done = lo_b >= hi_b
            mid = jnp.minimum((lo_b + hi_b) // 2, E - 1)
            blk = pl.multiple_of(mid & ~15, 16)
            lane = mid & 15
            pltpu.sync_copy(hidx_hbm.at[pl.ds(blk, 16)], probe_v)
            go_right = jnp.any((iota16 == lane) & (probe_v[...] < target))
            new_lo = jnp.where(done, lo_b, jnp.where(go_right, mid + 1, lo_b))
            new_hi = jnp.where(done, hi_b, jnp.where(go_right, hi_b, mid))
            return new_lo, new_hi
        ans, _ = lax.fori_loop(0, 18, step, (jnp.int32(0), jnp.int32(E)))
        return ans

    lo = first_geq(w * HPW)
    hi = first_geq((w + 1) * HPW)

    pltpu.sync_copy(zrow_hbm,
                    b_out.at[pl.ds(pl.multiple_of(w * HPW * NV, 8), HPW * NV)])

    def zero_row():
        def zstep(i, carry):
            row_v[pl.ds(i * 16, 16)] = jnp.zeros((16,), jnp.float32)
            return carry
        lax.fori_loop(0, NV // 16, zstep, 0)

    zero_row()

    def flush(h):
        pltpu.sync_copy(row_v.at[pl.ds(0, NV)],
                        b_out.at[pl.ds(pl.multiple_of(h * NV, 8), NV)])

    def chunk_body(ch, carry):
        base = ch * 128
        off = pl.multiple_of(jnp.maximum(base - 8, 0), 8)
        pltpu.sync_copy(nidx_hbm.at[pl.ds(off, 144)], nidx_v.at[pl.ds(0, 144)])
        pltpu.sync_copy(hidx_hbm.at[pl.ds(off, 144)], hidx_v.at[pl.ds(0, 144)])
        p0 = jnp.maximum(lo, base)
        p1 = jnp.minimum(hi, base + 128)

        def pair_body(p, carry):
            pos = p - off
            hv = hidx_v[pl.ds(pos, 16)]
            hprev = hidx_v[pl.ds(jnp.maximum(pos - 1, 0), 16)]
            changed = jnp.any((iota16 == 0) & (hv != hprev)) & (p > lo)

            @pl.when(changed)
            def _():
                flush(_extract(hprev, 11))
                zero_row()

            n = _extract(nidx_v[pl.ds(pos, 16)], 14)
            nlane = n & 15
            nb = n - nlane
            v = row_v[pl.ds(nb, 16)]
            row_v[pl.ds(nb, 16)] = v + jnp.where(iota16 == nlane, 1.0, 0.0)
            return carry

        return lax.fori_loop(p0, p1, pair_body, carry)

    lax.fori_loop(lo // 128, (hi + 127) // 128, chunk_body, 0)

    @pl.when(hi > lo)
    def _():
        last = hi - 1
        base = (last // 128) * 128
        off = jnp.maximum(base - 8, 0)
        flush(_extract(hidx_v[pl.ds(last - off, 16)], 11))


@functools.lru_cache(maxsize=None)
def _build_b():
  return pl.kernel(
    _build_b_body,
    out_type=jax.ShapeDtypeStruct((H * NV,), jnp.float32),
    mesh=_sc_mesh(),
    compiler_params=pltpu.CompilerParams(needs_layout_passes=False),
    scratch_types=[
        pltpu.VMEM((160,), jnp.int32),
        pltpu.VMEM((160,), jnp.int32),
        pltpu.VMEM((16,), jnp.int32),
        pltpu.VMEM((NV + 16,), jnp.float32),
    ],
  )


def _he_tc_body(b_ref, xw_ref, w_ref, bias_ref, out_ref):
    sa = jnp.dot(b_ref[...], xw_ref[...], preferred_element_type=jnp.float32)
    d = jnp.maximum(sa[:, D:D + 1], 1.0)
    mean = sa[:, :D] / d
    he = jnp.dot(mean, w_ref[...], preferred_element_type=jnp.float32) + bias_ref[...]
    he = jnp.maximum(he, 0.0)
    bh = he.shape[0]
    marker = (lax.broadcasted_iota(jnp.int32, (bh, W - D), 1) == 0)
    tail = jnp.where(marker, 1.0, 0.0).astype(jnp.float32)
    out_ref[...] = jnp.concatenate([he, tail], axis=1)


def _he_tc(b_mat, xw, w, bias):
    bh = 256
    return pl.pallas_call(
        _he_tc_body,
        grid=(H // bh,),
        in_specs=[
            pl.BlockSpec((bh, NV), lambda i: (i, 0)),
            pl.BlockSpec((NV, W), lambda i: (0, 0)),
            pl.BlockSpec((D, D), lambda i: (0, 0)),
            pl.BlockSpec((1, D), lambda i: (0, 0)),
        ],
        out_specs=pl.BlockSpec((bh, W), lambda i: (i, 0)),
        out_shape=jax.ShapeDtypeStruct((H, W), jnp.float32),
    )(b_mat, xw, w, bias)


def _final_tc_body(x_ref, b_ref, hea_ref, wself_ref, bself_ref,
                   we2v_ref, be2v_ref, w1_ref, b1_ref, w2_ref, b2_ref,
                   out_ref):
    sa = lax.dot_general(b_ref[...], hea_ref[...],
                         (((0,), (0,)), ((), ())),
                         preferred_element_type=jnp.float32)
    dv = jnp.maximum(sa[:, D:D + 1], 1.0)
    agg = sa[:, :D] / dv
    inc = jnp.dot(agg, we2v_ref[...], preferred_element_type=jnp.float32) + be2v_ref[...]
    xo = jnp.dot(x_ref[...], wself_ref[...], preferred_element_type=jnp.float32)
    xo = jnp.maximum(xo + bself_ref[...] + inc, 0.0)
    h = jnp.dot(xo, w1_ref[...], preferred_element_type=jnp.float32) + b1_ref[...]
    h = jnp.maximum(h, 0.0)
    lg = jnp.dot(h, w2_ref[...], preferred_element_type=jnp.float32) + b2_ref[...]
    m = jnp.max(lg, axis=1, keepdims=True)
    e = jnp.exp(lg - m)
    out_ref[...] = e / jnp.sum(e, axis=1, keepdims=True)


def _final_tc(x, b_mat, hea, w_self, b_self, w_e2v, b_e2v, w1, b1, w2, b2):
    bn = 256
    grid = (N + bn - 1) // bn
    return pl.pallas_call(
        _final_tc_body,
        grid=(grid,),
        in_specs=[
            pl.BlockSpec((bn, D), lambda i: (i, 0)),
            pl.BlockSpec((H, bn), lambda i: (0, i)),
            pl.BlockSpec((H, W), lambda i: (0, 0)),
            pl.BlockSpec((D, D), lambda i: (0, 0)),
            pl.BlockSpec((1, D), lambda i: (0, 0)),
            pl.BlockSpec((D, D), lambda i: (0, 0)),
            pl.BlockSpec((1, D), lambda i: (0, 0)),
            pl.BlockSpec((D, HID), lambda i: (0, 0)),
            pl.BlockSpec((1, HID), lambda i: (0, 0)),
            pl.BlockSpec((HID, C), lambda i: (0, 0)),
            pl.BlockSpec((1, C), lambda i: (0, 0)),
        ],
        out_specs=pl.BlockSpec((bn, C), lambda i: (i, 0)),
        out_shape=jax.ShapeDtypeStruct((N, C), jnp.float32),
    )(x, b_mat, hea, w_self, b_self, w_e2v, b_e2v, w1, b1, w2, b2)


def kernel(x, node_idx, hedge_idx, W_v2e, b_v2e, W_e2v, b_e2v,
           W_self, b_self, W1, b1, W2, b2):
    x = x.astype(jnp.float32)
    ni = node_idx.astype(jnp.int32)
    hi = hedge_idx.astype(jnp.int32)
    xw = jnp.concatenate(
        [x, jnp.ones((N, 1), jnp.float32),
         jnp.zeros((N, W - D - 1), jnp.float32)], axis=1)
    xwp = jnp.concatenate([xw, jnp.zeros((NV - N, W), jnp.float32)], axis=0)
    zrow = jnp.zeros((HPW * NV,), jnp.float32)

    b_flat = _build_b()(ni, hi, zrow)
    b_mat = b_flat.reshape(H, NV)
    hea = _he_tc(b_mat, xwp, W_v2e, b_v2e.reshape(1, D))
    return _final_tc(x, b_mat, hea, W_self, b_self.reshape(1, D),
                     W_e2v, b_e2v.reshape(1, D), W1, b1.reshape(1, HID),
                     W2, b2.reshape(1, C))


# speedup vs baseline: 1.0994x; 1.0994x over previous
import functools

import jax
import jax.numpy as jnp
from jax import lax
from jax.experimental import pallas as pl
from jax.experimental.pallas import tpu as pltpu
from jax.experimental.pallas import tpu_sc as plsc

N = 10000
E = 160000
D = 256
H = 2048
HID = 128
C = 40

W = 384
NC = 2
NS = 16
NW = NC * NS
HPW = H // NW
NV = 10240
NB16 = E // 16


@functools.lru_cache(maxsize=None)
def _sc_mesh():
    return plsc.VectorSubcoreMesh(core_axis_name="c", subcore_axis_name="s",
                                  num_cores=NC, num_subcores=NS)


def _extract(v, nbits, lane):
    i = lax.iota(jnp.int32, 16)
    sel = i == lane
    acc = jnp.int32(0)
    for k in range(nbits):
        bit = jnp.any(sel & (((v >> k) & 1) == 1))
        acc = acc + jnp.where(bit, jnp.int32(1 << k), jnp.int32(0))
    return acc


def _build_b_body(nidx_hbm, hidx_hbm, zrow_hbm, b_out,
                  nidx_v, hidx_v, probe_v, row_v):
    c = lax.axis_index("c")
    s = lax.axis_index("s")
    w = c * NS + s
    iota16 = lax.iota(jnp.int32, 16)

    def first_geq(target):
        def step(_, carry):
            lo_b, hi_b = carry
            done = lo_b >= hi_b
            mid = jnp.minimum((lo_b + hi_b) // 2, E - 1)
            blk = pl.multiple_of(mid & ~15, 16)
            lane = mid & 15
            pltpu.sync_copy(hidx_hbm.at[pl.ds(blk, 16)], probe_v)
            go_right = jnp.any((iota16 == lane) & (probe_v[...] < target))
            new_lo = jnp.where(done, lo_b, jnp.where(go_right, mid + 1, lo_b))
            new_hi = jnp.where(done, hi_b, jnp.where(go_right, hi_b, mid))
            return new_lo, new_hi
        ans, _ = lax.fori_loop(0, 18, step, (jnp.int32(0), jnp.int32(E)))
        return ans

    lo = first_geq(w * HPW)
    hi = first_geq((w + 1) * HPW)

    pltpu.sync_copy(zrow_hbm,
                    b_out.at[pl.ds(pl.multiple_of(w * HPW * NV, 8), HPW * NV)])

    def zero_row():
        def zstep(i, carry):
            row_v[pl.ds(i * 16, 16)] = jnp.zeros((16,), jnp.float32)
            return carry
        lax.fori_loop(0, NV // 16, zstep, 0)

    zero_row()

    def flush(h):
        pltpu.sync_copy(row_v.at[pl.ds(0, NV)],
                        b_out.at[pl.ds(pl.multiple_of(h * NV, 8), NV)])

    def chunk_body(ch, carry):
        base = ch * 128
        off = pl.multiple_of(jnp.maximum(base - 8, 0), 8)
        pltpu.sync_copy(nidx_hbm.at[pl.ds(off, 144)], nidx_v.at[pl.ds(0, 144)])
        pltpu.sync_copy(hidx_hbm.at[pl.ds(off, 144)], hidx_v.at[pl.ds(0, 144)])

        for gi in range(8):
            pg = base + gi * 16
            pos = pg - off
            hv = hidx_v[pl.ds(pos, 16)]
            nv = nidx_v[pl.ds(pos, 16)]
            hprev = hidx_v[pl.ds(jnp.maximum(pos - 1, 0), 16)]
            valid = (iota16 >= lo - pg) & (iota16 < hi - pg)
            prevgate = (iota16 + (pg - lo)) > 0
            change = (hv != hprev) & valid & prevgate
            has_boundary = jnp.any(change)

            @pl.when(jnp.logical_not(has_boundary))
            def _():
                plsc.addupdate_scatter(row_v, [nv],
                                       jnp.where(valid, 1.0, 0.0))

            @pl.when(has_boundary)
            def _():
                def lane_body(l, carry):
                    ch_l = jnp.any(change & (iota16 == l))

                    @pl.when(ch_l)
                    def _():
                        flush(_extract(hprev, 11, l))
                        zero_row()

                    plsc.addupdate_scatter(
                        row_v, [nv], jnp.where(iota16 == l, 1.0, 0.0))
                    return carry

                l0 = jnp.maximum(lo - pg, 0)
                l1 = jnp.minimum(hi - pg, 16)
                lax.fori_loop(l0, l1, lane_body, 0)
        return carry

    lax.fori_loop(lo // 128, (hi + 127) // 128, chunk_body, 0)

    @pl.when(hi > lo)
    def _():
        last = hi - 1
        base = (last // 128) * 128
        off = jnp.maximum(base - 8, 0)
        flush(_extract(hidx_v[pl.ds(last - off, 16)], 11, 0))


@functools.lru_cache(maxsize=None)
def _build_b():
  return pl.kernel(
    _build_b_body,
    out_type=jax.ShapeDtypeStruct((H * NV,), jnp.float32),
    mesh=_sc_mesh(),
    compiler_params=pltpu.CompilerParams(needs_layout_passes=False),
    scratch_types=[
        pltpu.VMEM((160,), jnp.int32),
        pltpu.VMEM((160,), jnp.int32),
        pltpu.VMEM((16,), jnp.int32),
        pltpu.VMEM((NV + 16,), jnp.float32),
    ],
  )


def _he_tc_body(b_ref, xw_ref, w_ref, bias_ref, out_ref):
    sa = jnp.dot(b_ref[...], xw_ref[...], preferred_element_type=jnp.float32)
    d = jnp.maximum(sa[:, D:D + 1], 1.0)
    mean = sa[:, :D] / d
    he = jnp.dot(mean, w_ref[...], preferred_element_type=jnp.float32) + bias_ref[...]
    he = jnp.maximum(he, 0.0)
    bh = he.shape[0]
    marker = (lax.broadcasted_iota(jnp.int32, (bh, W - D), 1) == 0)
    tail = jnp.where(marker, 1.0, 0.0).astype(jnp.float32)
    out_ref[...] = jnp.concatenate([he, tail], axis=1)


def _he_tc(b_mat, xw, w, bias):
    bh = 256
    return pl.pallas_call(
        _he_tc_body,
        grid=(H // bh,),
        in_specs=[
            pl.BlockSpec((bh, NV), lambda i: (i, 0)),
            pl.BlockSpec((NV, W), lambda i: (0, 0)),
            pl.BlockSpec((D, D), lambda i: (0, 0)),
            pl.BlockSpec((1, D), lambda i: (0, 0)),
        ],
        out_specs=pl.BlockSpec((bh, W), lambda i: (i, 0)),
        out_shape=jax.ShapeDtypeStruct((H, W), jnp.float32),
    )(b_mat, xw, w, bias)


def _final_tc_body(x_ref, b_ref, hea_ref, wself_ref, bself_ref,
                   we2v_ref, be2v_ref, w1_ref, b1_ref, w2_ref, b2_ref,
                   out_ref):
    sa = lax.dot_general(b_ref[...], hea_ref[...],
                         (((0,), (0,)), ((), ())),
                         preferred_element_type=jnp.float32)
    dv = jnp.maximum(sa[:, D:D + 1], 1.0)
    agg = sa[:, :D] / dv
    inc = jnp.dot(agg, we2v_ref[...], preferred_element_type=jnp.float32) + be2v_ref[...]
    xo = jnp.dot(x_ref[...], wself_ref[...], preferred_element_type=jnp.float32)
    xo = jnp.maximum(xo + bself_ref[...] + inc, 0.0)
    h = jnp.dot(xo, w1_ref[...], preferred_element_type=jnp.float32) + b1_ref[...]
    h = jnp.maximum(h, 0.0)
    lg = jnp.dot(h, w2_ref[...], preferred_element_type=jnp.float32) + b2_ref[...]
    m = jnp.max(lg, axis=1, keepdims=True)
    e = jnp.exp(lg - m)
    out_ref[...] = e / jnp.sum(e, axis=1, keepdims=True)


def _final_tc(x, b_mat, hea, w_self, b_self, w_e2v, b_e2v, w1, b1, w2, b2):
    bn = 256
    grid = (N + bn - 1) // bn
    return pl.pallas_call(
        _final_tc_body,
        grid=(grid,),
        in_specs=[
            pl.BlockSpec((bn, D), lambda i: (i, 0)),
            pl.BlockSpec((H, bn), lambda i: (0, i)),
            pl.BlockSpec((H, W), lambda i: (0, 0)),
            pl.BlockSpec((D, D), lambda i: (0, 0)),
            pl.BlockSpec((1, D), lambda i: (0, 0)),
            pl.BlockSpec((D, D), lambda i: (0, 0)),
            pl.BlockSpec((1, D), lambda i: (0, 0)),
            pl.BlockSpec((D, HID), lambda i: (0, 0)),
            pl.BlockSpec((1, HID), lambda i: (0, 0)),
            pl.BlockSpec((HID, C), lambda i: (0, 0)),
            pl.BlockSpec((1, C), lambda i: (0, 0)),
        ],
        out_specs=pl.BlockSpec((bn, C), lambda i: (i, 0)),
        out_shape=jax.ShapeDtypeStruct((N, C), jnp.float32),
    )(x, b_mat, hea, w_self, b_self, w_e2v, b_e2v, w1, b1, w2, b2)


def kernel(x, node_idx, hedge_idx, W_v2e, b_v2e, W_e2v, b_e2v,
           W_self, b_self, W1, b1, W2, b2):
    x = x.astype(jnp.float32)
    ni = node_idx.astype(jnp.int32)
    hi = hedge_idx.astype(jnp.int32)
    xw = jnp.concatenate(
        [x, jnp.ones((N, 1), jnp.float32),
         jnp.zeros((N, W - D - 1), jnp.float32)], axis=1)
    xwp = jnp.concatenate([xw, jnp.zeros((NV - N, W), jnp.float32)], axis=0)
    zrow = jnp.zeros((HPW * NV,), jnp.float32)

    b_flat = _build_b()(ni, hi, zrow)
    b_mat = b_flat.reshape(H, NV)
    hea = _he_tc(b_mat, xwp, W_v2e, b_v2e.reshape(1, D))
    return _final_tc(x, b_mat, hea, W_self, b_self.reshape(1, D),
                     W_e2v, b_e2v.reshape(1, D), W1, b1.reshape(1, HID),
                     W2, b2.reshape(1, C))


# speedup vs baseline: 1.1832x; 1.0762x over previous
import functools

import jax
import jax.numpy as jnp
from jax import lax
from jax.experimental import pallas as pl
from jax.experimental.pallas import tpu as pltpu
from jax.experimental.pallas import tpu_sc as plsc

N = 10000
E = 160000
D = 256
H = 2048
HID = 128
C = 40

W = 384
NC = 2
NS = 16
NW = NC * NS
HPW = H // NW
NV = 10240
CHW = 1024


@functools.lru_cache(maxsize=None)
def _sc_mesh():
    return plsc.VectorSubcoreMesh(core_axis_name="c", subcore_axis_name="s",
                                  num_cores=NC, num_subcores=NS)


def _extract(v, nbits, lane):
    i = lax.iota(jnp.int32, 16)
    sel = i == lane
    acc = jnp.int32(0)
    for k in range(nbits):
        bit = jnp.any(sel & (((v >> k) & 1) == 1))
        acc = acc + jnp.where(bit, jnp.int32(1 << k), jnp.int32(0))
    return acc


def _build_b_body(nidx_hbm, hidx_hbm, zrow_hbm, b_out,
                  nidx_v, hidx_v, probe_v, acc_v):
    c = lax.axis_index("c")
    s = lax.axis_index("s")
    w = c * NS + s
    iota16 = lax.iota(jnp.int32, 16)

    def first_geq(target):
        def step(_, carry):
            lo_b, hi_b = carry
            done = lo_b >= hi_b
            mid = jnp.minimum((lo_b + hi_b) // 2, E - 1)
            blk = pl.multiple_of(mid & ~15, 16)
            lane = mid & 15
            pltpu.sync_copy(hidx_hbm.at[pl.ds(blk, 16)], probe_v)
            go_right = jnp.any((iota16 == lane) & (probe_v[...] < target))
            new_lo = jnp.where(done, lo_b, jnp.where(go_right, mid + 1, lo_b))
            new_hi = jnp.where(done, hi_b, jnp.where(go_right, hi_b, mid))
            return new_lo, new_hi
        ans, _ = lax.fori_loop(0, 18, step, (jnp.int32(0), jnp.int32(E)))
        return ans

    lo = first_geq(w * HPW)
    hi = first_geq((w + 1) * HPW)

    pltpu.sync_copy(zrow_hbm,
                    b_out.at[pl.ds(pl.multiple_of(w * HPW * NV, 8), HPW * NV)])

    def zero_acc():
        pltpu.sync_copy(zrow_hbm.at[pl.ds(0, 8 * NV)], acc_v)

    zero_acc()

    def flush(blk):
        pltpu.sync_copy(acc_v,
                        b_out.at[pl.ds(pl.multiple_of(blk * 8 * NV, 8), 8 * NV)])

    def chunk_body(ch, carry):
        base = ch * CHW
        off = pl.multiple_of(
            jnp.minimum(jnp.maximum(base - 8, 0), E - CHW - 8), 8)
        pltpu.sync_copy(nidx_hbm.at[pl.ds(off, CHW + 8)],
                        nidx_v.at[pl.ds(0, CHW + 8)])
        pltpu.sync_copy(hidx_hbm.at[pl.ds(off, CHW + 8)],
                        hidx_v.at[pl.ds(0, CHW + 8)])
        g0 = jnp.maximum((lo - base) // 16, 0)
        g1 = jnp.minimum((hi - base + 15) // 16, CHW // 16)

        def group_body(gi, carry):
            pg = base + gi * 16
            pos = pg - off
            hv = hidx_v[pl.ds(pos, 16)]
            nv = nidx_v[pl.ds(pos, 16)]
            hprev = hidx_v[pl.ds(jnp.maximum(pos - 1, 0), 16)]
            valid = (iota16 >= lo - pg) & (iota16 < hi - pg)
            prevgate = (iota16 + (pg - lo)) > 0
            change = ((hv >> 3) != (hprev >> 3)) & valid & prevgate
            has_boundary = jnp.any(change)
            flat = (hv & 7) * NV + nv

            @pl.when(jnp.logical_not(has_boundary))
            def _():
                plsc.addupdate_scatter(acc_v, [flat],
                                       jnp.where(valid, 1.0, 0.0))

            @pl.when(has_boundary)
            def _():
                def lane_body(l, carry):
                    ch_l = jnp.any(change & (iota16 == l))

                    @pl.when(ch_l)
                    def _():
                        flush(_extract(hprev >> 3, 8, l))
                        zero_acc()

                    plsc.addupdate_scatter(
                        acc_v, [flat], jnp.where(iota16 == l, 1.0, 0.0))
                    return carry

                l0 = jnp.maximum(lo - pg, 0)
                l1 = jnp.minimum(hi - pg, 16)
                lax.fori_loop(l0, l1, lane_body, 0)
            return carry

        return lax.fori_loop(g0, g1, group_body, carry)

    lax.fori_loop(lo // CHW, (hi + CHW - 1) // CHW, chunk_body, 0)

    @pl.when(hi > lo)
    def _():
        last = hi - 1
        base = (last // CHW) * CHW
        off = jnp.minimum(jnp.maximum(base - 8, 0), E - CHW - 8)
        flush(_extract(hidx_v[pl.ds(last - off, 16)] >> 3, 8, 0))


@functools.lru_cache(maxsize=None)
def _build_b():
  return pl.kernel(
    _build_b_body,
    out_type=jax.ShapeDtypeStruct((H * NV,), jnp.float32),
    mesh=_sc_mesh(),
    compiler_params=pltpu.CompilerParams(needs_layout_passes=False),
    scratch_types=[
        pltpu.VMEM((CHW + 32,), jnp.int32),
        pltpu.VMEM((CHW + 32,), jnp.int32),
        pltpu.VMEM((16,), jnp.int32),
        pltpu.VMEM((8 * NV,), jnp.float32),
    ],
  )


def _he_tc_body(b_ref, xw_ref, w_ref, bias_ref, out_ref):
    sa = jnp.dot(b_ref[...], xw_ref[...], preferred_element_type=jnp.float32)
    d = jnp.maximum(sa[:, D:D + 1], 1.0)
    mean = sa[:, :D] / d
    he = jnp.dot(mean, w_ref[...], preferred_element_type=jnp.float32) + bias_ref[...]
    he = jnp.maximum(he, 0.0)
    bh = he.shape[0]
    marker = (lax.broadcasted_iota(jnp.int32, (bh, W - D), 1) == 0)
    tail = jnp.where(marker, 1.0, 0.0).astype(jnp.float32)
    out_ref[...] = jnp.concatenate([he, tail], axis=1)


def _he_tc(b_mat, xw, w, bias):
    bh = 256
    return pl.pallas_call(
        _he_tc_body,
        grid=(H // bh,),
        in_specs=[
            pl.BlockSpec((bh, NV), lambda i: (i, 0)),
            pl.BlockSpec((NV, W), lambda i: (0, 0)),
            pl.BlockSpec((D, D), lambda i: (0, 0)),
            pl.BlockSpec((1, D), lambda i: (0, 0)),
        ],
        out_specs=pl.BlockSpec((bh, W), lambda i: (i, 0)),
        out_shape=jax.ShapeDtypeStruct((H, W), jnp.float32),
    )(b_mat, xw, w, bias)


def _final_tc_body(x_ref, b_ref, hea_ref, wself_ref, bself_ref,
                   we2v_ref, be2v_ref, w1_ref, b1_ref, w2_ref, b2_ref,
                   out_ref):
    sa = lax.dot_general(b_ref[...], hea_ref[...],
                         (((0,), (0,)), ((), ())),
                         preferred_element_type=jnp.float32)
    dv = jnp.maximum(sa[:, D:D + 1], 1.0)
    agg = sa[:, :D] / dv
    inc = jnp.dot(agg, we2v_ref[...], preferred_element_type=jnp.float32) + be2v_ref[...]
    xo = jnp.dot(x_ref[...], wself_ref[...], preferred_element_type=jnp.float32)
    xo = jnp.maximum(xo + bself_ref[...] + inc, 0.0)
    h = jnp.dot(xo, w1_ref[...], preferred_element_type=jnp.float32) + b1_ref[...]
    h = jnp.maximum(h, 0.0)
    lg = jnp.dot(h, w2_ref[...], preferred_element_type=jnp.float32) + b2_ref[...]
    m = jnp.max(lg, axis=1, keepdims=True)
    e = jnp.exp(lg - m)
    out_ref[...] = e / jnp.sum(e, axis=1, keepdims=True)


def _final_tc(x, b_mat, hea, w_self, b_self, w_e2v, b_e2v, w1, b1, w2, b2):
    bn = 256
    grid = (N + bn - 1) // bn
    return pl.pallas_call(
        _final_tc_body,
        grid=(grid,),
        in_specs=[
            pl.BlockSpec((bn, D), lambda i: (i, 0)),
            pl.BlockSpec((H, bn), lambda i: (0, i)),
            pl.BlockSpec((H, W), lambda i: (0, 0)),
            pl.BlockSpec((D, D), lambda i: (0, 0)),
            pl.BlockSpec((1, D), lambda i: (0, 0)),
            pl.BlockSpec((D, D), lambda i: (0, 0)),
            pl.BlockSpec((1, D), lambda i: (0, 0)),
            pl.BlockSpec((D, HID), lambda i: (0, 0)),
            pl.BlockSpec((1, HID), lambda i: (0, 0)),
            pl.BlockSpec((HID, C), lambda i: (0, 0)),
            pl.BlockSpec((1, C), lambda i: (0, 0)),
        ],
        out_specs=pl.BlockSpec((bn, C), lambda i: (i, 0)),
        out_shape=jax.ShapeDtypeStruct((N, C), jnp.float32),
    )(x, b_mat, hea, w_self, b_self, w_e2v, b_e2v, w1, b1, w2, b2)


def kernel(x, node_idx, hedge_idx, W_v2e, b_v2e, W_e2v, b_e2v,
           W_self, b_self, W1, b1, W2, b2):
    x = x.astype(jnp.float32)
    ni = node_idx.astype(jnp.int32)
    hi = hedge_idx.astype(jnp.int32)
    xw = jnp.concatenate(
        [x, jnp.ones((N, 1), jnp.float32),
         jnp.zeros((N, W - D - 1), jnp.float32)], axis=1)
    xwp = jnp.concatenate([xw, jnp.zeros((NV - N, W), jnp.float32)], axis=0)
    zrow = jnp.zeros((HPW * NV,), jnp.float32)

    b_flat = _build_b()(ni, hi, zrow)
    b_mat = b_flat.reshape(H, NV)
    hea = _he_tc(b_mat, xwp, W_v2e, b_v2e.reshape(1, D))
    return _final_tc(x, b_mat, hea, W_self, b_self.reshape(1, D),
                     W_e2v, b_e2v.reshape(1, D), W1, b1.reshape(1, HID),
                     W2, b2.reshape(1, C))
